# baseline (speedup 1.0000x reference)
import jax
import jax.numpy as jnp
from jax import lax
from jax.experimental import pallas as pl
from jax.experimental.pallas import tpu as pltpu

EMB = 128
BLK = 16384


def _body(flag_l_ref, flag_r_ref, w_ref, b_ref, xl_in, xr_in,
          out_l_ref, out_r_ref, xl_out, xr_out, sem_l, sem_r):
    i = pl.program_id(0)

    @pl.when(i == 0)
    def _start_passthrough():
        pltpu.make_async_copy(xl_in, xl_out, sem_l).start()
        pltpu.make_async_copy(xr_in, xr_out, sem_r).start()

    w = w_ref[...]
    b = b_ref[...]
    dn = (((0,), (0,)), ((), ()))
    fl = flag_l_ref[...].reshape(1, BLK)
    fr = flag_r_ref[...].reshape(1, BLK)
    out_l_ref[...] = lax.dot_general(
        fl, w, dn, preferred_element_type=jnp.float32) + b
    out_r_ref[...] = lax.dot_general(
        fr, w, dn, preferred_element_type=jnp.float32) + b

    @pl.when(i == pl.num_programs(0) - 1)
    def _finish_passthrough():
        pltpu.make_async_copy(xl_in, xl_out, sem_l).wait()
        pltpu.make_async_copy(xr_in, xr_out, sem_r).wait()


def kernel(x_lig, x_rec, v_lig, v_rec, aa_rec, batch_idx_lig, batch_idx_rec,
           lig_flag, rec_flag, W_ind, b_ind):
    n = lig_flag.shape[0]
    w_row = W_ind.reshape(1, EMB)
    b_row = b_ind.reshape(1, EMB)

    h_lig, h_rec, xl_out, xr_out = pl.pallas_call(
        _body,
        grid=(pl.cdiv(n, BLK),),
        in_specs=[
            pl.BlockSpec((BLK,), lambda i: (i,)),
            pl.BlockSpec((BLK,), lambda i: (i,)),
            pl.BlockSpec((1, EMB), lambda i: (0, 0)),
            pl.BlockSpec((1, EMB), lambda i: (0, 0)),
            pl.BlockSpec(memory_space=pltpu.MemorySpace.HBM),
            pl.BlockSpec(memory_space=pltpu.MemorySpace.HBM),
        ],
        out_specs=[
            pl.BlockSpec((BLK, EMB), lambda i: (i, 0)),
            pl.BlockSpec((BLK, EMB), lambda i: (i, 0)),
            pl.BlockSpec(memory_space=pltpu.MemorySpace.HBM),
            pl.BlockSpec(memory_space=pltpu.MemorySpace.HBM),
        ],
        out_shape=[
            jax.ShapeDtypeStruct((n, EMB), jnp.float32),
            jax.ShapeDtypeStruct((n, EMB), jnp.float32),
            jax.ShapeDtypeStruct(x_lig.shape, x_lig.dtype),
            jax.ShapeDtypeStruct(x_rec.shape, x_rec.dtype),
        ],
        scratch_shapes=[pltpu.SemaphoreType.DMA, pltpu.SemaphoreType.DMA],
    )(lig_flag, rec_flag, w_row, b_row, x_lig, x_rec)

    return (xl_out, xr_out, h_lig, h_rec)


# speedup vs baseline: 16.1620x; 16.1620x over previous
import jax
import jax.numpy as jnp
from jax import lax
from jax.experimental import pallas as pl

EMB = 128
BLK = 8192


def _body(flag_l_ref, flag_r_ref, w_ref, b_ref, xl_in, xr_in,
          out_l_ref, out_r_ref, xl_out, xr_out):
    w = w_ref[...]
    b = b_ref[...]
    dn = (((0,), (0,)), ((), ()))
    fl = flag_l_ref[...].reshape(1, BLK)
    fr = flag_r_ref[...].reshape(1, BLK)
    out_l_ref[...] = lax.dot_general(
        fl, w, dn, preferred_element_type=jnp.float32) + b
    out_r_ref[...] = lax.dot_general(
        fr, w, dn, preferred_element_type=jnp.float32) + b
    xl_out[...] = xl_in[...]
    xr_out[...] = xr_in[...]


def kernel(x_lig, x_rec, v_lig, v_rec, aa_rec, batch_idx_lig, batch_idx_rec,
           lig_flag, rec_flag, W_ind, b_ind):
    n = lig_flag.shape[0]
    w_row = W_ind.reshape(1, EMB)
    b_row = b_ind.reshape(1, EMB)

    h_lig, h_rec, xl_out, xr_out = pl.pallas_call(
        _body,
        grid=(pl.cdiv(n, BLK),),
        in_specs=[
            pl.BlockSpec((BLK,), lambda i: (i,)),
            pl.BlockSpec((BLK,), lambda i: (i,)),
            pl.BlockSpec((1, EMB), lambda i: (0, 0)),
            pl.BlockSpec((1, EMB), lambda i: (0, 0)),
            pl.BlockSpec((BLK, 3), lambda i: (i, 0)),
            pl.BlockSpec((BLK, 3), lambda i: (i, 0)),
        ],
        out_specs=[
            pl.BlockSpec((BLK, EMB), lambda i: (i, 0)),
            pl.BlockSpec((BLK, EMB), lambda i: (i, 0)),
            pl.BlockSpec((BLK, 3), lambda i: (i, 0)),
            pl.BlockSpec((BLK, 3), lambda i: (i, 0)),
        ],
        out_shape=[
            jax.ShapeDtypeStruct((n, EMB), jnp.float32),
            jax.ShapeDtypeStruct((n, EMB), jnp.float32),
            jax.ShapeDtypeStruct(x_lig.shape, x_lig.dtype),
            jax.ShapeDtypeStruct(x_rec.shape, x_rec.dtype),
        ],
    )(lig_flag, rec_flag, w_row, b_row, x_lig, x_rec)

    return (xl_out, xr_out, h_lig, h_rec)


# speedup vs baseline: 74.6827x; 4.6209x over previous
import jax
import jax.numpy as jnp
from jax.experimental import pallas as pl

EMB = 128
BLK = 16384


def _fill_body(flag_l_ref, flag_r_ref, w_ref, b_ref, out_l_ref, out_r_ref):
    w = w_ref[...]
    b = b_ref[...]
    dn = (((0,), (0,)), ((), ()))
    fl = flag_l_ref[...].reshape(1, BLK)
    fr = flag_r_ref[...].reshape(1, BLK)
    out_l_ref[...] = jax.lax.dot_general(
        fl, w, dn, preferred_element_type=jnp.float32) + b
    out_r_ref[...] = jax.lax.dot_general(
        fr, w, dn, preferred_element_type=jnp.float32) + b


def kernel(x_lig, x_rec, v_lig, v_rec, aa_rec, batch_idx_lig, batch_idx_rec,
           lig_flag, rec_flag, W_ind, b_ind):
    n_lig = lig_flag.shape[0]
    n_rec = rec_flag.shape[0]
    assert n_lig == n_rec
    n = n_lig
    grid = (pl.cdiv(n, BLK),)

    flag_l = lig_flag
    flag_r = rec_flag
    w_row = W_ind.reshape(1, EMB)
    b_row = b_ind.reshape(1, EMB)

    h_lig, h_rec = pl.pallas_call(
        _fill_body,
        grid=grid,
        in_specs=[
            pl.BlockSpec((BLK,), lambda i: (i,)),
            pl.BlockSpec((BLK,), lambda i: (i,)),
            pl.BlockSpec((1, EMB), lambda i: (0, 0)),
            pl.BlockSpec((1, EMB), lambda i: (0, 0)),
        ],
        out_specs=[
            pl.BlockSpec((BLK, EMB), lambda i: (i, 0)),
            pl.BlockSpec((BLK, EMB), lambda i: (i, 0)),
        ],
        out_shape=[
            jax.ShapeDtypeStruct((n, EMB), jnp.float32),
            jax.ShapeDtypeStruct((n, EMB), jnp.float32),
        ],
    )(flag_l, flag_r, w_row, b_row)

    return (x_lig, x_rec, h_lig, h_rec)
